# Initial kernel scaffold; baseline (speedup 1.0000x reference)
import jax
import jax.numpy as jnp
from jax.experimental import pallas as pl


def kernel(features, labels, W):
    raise NotImplementedError("write your pallas kernel here")


# speedup vs baseline: 1.7496x; 1.7496x over previous
import functools

import jax
import jax.numpy as jnp
from jax import lax
from jax.experimental import pallas as pl
from jax.experimental.pallas import tpu as pltpu
from jax.experimental.pallas import tpu_sc as plsc

B = 16384
C = 1000
D = 128

NC = 2
NS = 16
L = 16
NW = NC * NS

CHUNK = 128
CHUNKS_PER_W = B // (NW * CHUNK)


@functools.partial(
    pl.kernel,
    mesh=plsc.VectorSubcoreMesh(core_axis_name="c", subcore_axis_name="s"),
    out_type=jax.ShapeDtypeStruct((B, D), jnp.float32),
    scratch_types=[
        pltpu.VMEM((CHUNKS_PER_W, CHUNK), jnp.int32),
        pltpu.VMEM((CHUNK, D), jnp.float32),
        pltpu.VMEM((CHUNK, D), jnp.float32),
        pltpu.SemaphoreType.DMA,
    ],
)
def _sc_weighter(feat_hbm, lab_hbm, w_hbm, out_hbm, idx_v, rows_v, feat_v, sem):
    wid = lax.axis_index("s") * NC + lax.axis_index("c")
    pltpu.sync_copy(lab_hbm.at[wid], idx_v)

    for j in range(CHUNKS_PER_W):
        row0 = (wid * CHUNKS_PER_W + j) * CHUNK
        pltpu.async_copy(w_hbm.at[idx_v.at[j]], rows_v, sem)
        pltpu.sync_copy(feat_hbm.at[pl.ds(row0, CHUNK)], feat_v)
        pltpu.make_async_copy(w_hbm.at[idx_v.at[j]], rows_v, sem).wait()

        def row_body(r, carry):
            for c in range(D // L):
                w = rows_v[r, pl.ds(c * L, L)]
                f = feat_v[r, pl.ds(c * L, L)]
                feat_v[r, pl.ds(c * L, L)] = f / (1.0 + jnp.exp(-w))
            return carry

        lax.fori_loop(0, CHUNK, row_body, 0)
        pltpu.sync_copy(feat_v, out_hbm.at[pl.ds(row0, CHUNK)])


def kernel(features, labels, W):
    lab = labels.astype(jnp.int32).reshape(NW, CHUNKS_PER_W, CHUNK)
    return _sc_weighter(features, lab, W)


# speedup vs baseline: 1.9321x; 1.1043x over previous
import functools

import jax
import jax.numpy as jnp
from jax import lax
from jax.experimental import pallas as pl
from jax.experimental.pallas import tpu as pltpu
from jax.experimental.pallas import tpu_sc as plsc

B = 16384
C = 1000
D = 128

NC = 2
NS = 16
L = 16
NW = NC * NS

CHUNK = 128
NCHUNK = B // (NW * CHUNK)
NBUF = 2


@functools.partial(
    pl.kernel,
    mesh=plsc.VectorSubcoreMesh(core_axis_name="c", subcore_axis_name="s"),
    out_type=jax.ShapeDtypeStruct((B, D), jnp.float32),
    scratch_types=(
        [pltpu.VMEM((NCHUNK, CHUNK), jnp.int32)]
        + [pltpu.VMEM((CHUNK, D), jnp.float32) for _ in range(3 * NBUF)]
        + [pltpu.SemaphoreType.DMA for _ in range(3 * NBUF)]
    ),
)
def _sc_weighter(feat_hbm, lab_hbm, w_hbm, out_hbm,
                 idx_v, rows0, rows1, feat0, feat1, out0, out1,
                 sg0, sg1, sf0, sf1, so0, so1):
    rows_v = (rows0, rows1)
    feat_v = (feat0, feat1)
    out_v = (out0, out1)
    sem_g = (sg0, sg1)
    sem_f = (sf0, sf1)
    sem_o = (so0, so1)

    wid = lax.axis_index("s") * NC + lax.axis_index("c")
    pltpu.sync_copy(lab_hbm.at[wid], idx_v)

    def row0_of(j):
        return (wid * NCHUNK + j) * CHUNK

    gathers = {}
    feats = {}
    outs = {}

    def issue_loads(j):
        p = j % NBUF
        gathers[j] = pltpu.async_copy(w_hbm.at[idx_v.at[j]], rows_v[p], sem_g[p])
        feats[j] = pltpu.async_copy(feat_hbm.at[pl.ds(row0_of(j), CHUNK)],
                                    feat_v[p], sem_f[p])

    issue_loads(0)
    issue_loads(1)

    for j in range(NCHUNK):
        p = j % NBUF
        gathers[j].wait()
        feats[j].wait()
        if j >= NBUF:
            outs[j - NBUF].wait()

        def row_body(r, carry):
            for c in range(D // L):
                w = rows_v[p][r, pl.ds(c * L, L)]
                f = feat_v[p][r, pl.ds(c * L, L)]
                out_v[p][r, pl.ds(c * L, L)] = f / (1.0 + jnp.exp(-w))
            return carry

        lax.fori_loop(0, CHUNK, row_body, 0)
        outs[j] = pltpu.async_copy(out_v[p], out_hbm.at[pl.ds(row0_of(j), CHUNK)],
                                   sem_o[p])
        if j + NBUF < NCHUNK:
            issue_loads(j + NBUF)

    for j in range(NCHUNK - NBUF, NCHUNK):
        outs[j].wait()


def kernel(features, labels, W):
    lab = labels.astype(jnp.int32).reshape(NW, NCHUNK, CHUNK)
    return _sc_weighter(features, lab, W)


# speedup vs baseline: 2.0679x; 1.0703x over previous
import functools

import jax
import jax.numpy as jnp
from jax import lax
from jax.experimental import pallas as pl
from jax.experimental.pallas import tpu as pltpu
from jax.experimental.pallas import tpu_sc as plsc

B = 16384
C = 1000
D = 128

NC = 2
NS = 16
L = 16
NW = NC * NS

CP = 1024
WROWS = CP // NS
CHUNK = 128
NCHUNK = B // (NW * CHUNK)
NBUF = 2


@functools.partial(
    pl.kernel,
    mesh=plsc.VectorSubcoreMesh(core_axis_name="c", subcore_axis_name="s"),
    out_type=jax.ShapeDtypeStruct((B, D), jnp.float32),
    scratch_types=(
        [pltpu.VMEM((NCHUNK, CHUNK), jnp.int32),
         pltpu.VMEM((WROWS, D), jnp.float32),
         pltpu.VMEM_SHARED((CP, D), jnp.float32)]
        + [pltpu.VMEM((CHUNK, D), jnp.float32) for _ in range(3 * NBUF)]
        + [pltpu.SemaphoreType.DMA for _ in range(3 * NBUF)]
    ),
)
def _sc_weighter(feat_hbm, lab_hbm, w_hbm, out_hbm,
                 idx_v, wtile_v, sig_sh,
                 rows0, rows1, feat0, feat1, out0, out1,
                 sg0, sg1, sf0, sf1, so0, so1):
    rows_v = (rows0, rows1)
    feat_v = (feat0, feat1)
    out_v = (out0, out1)
    sem_g = (sg0, sg1)
    sem_f = (sf0, sf1)
    sem_o = (so0, so1)

    sid = lax.axis_index("s")
    wid = sid * NC + lax.axis_index("c")
    pltpu.sync_copy(lab_hbm.at[wid], idx_v)

    def row0_of(j):
        return (wid * NCHUNK + j) * CHUNK

    feats = {}
    outs = {}

    def issue_feat(j):
        p = j % NBUF
        feats[j] = pltpu.async_copy(feat_hbm.at[pl.ds(row0_of(j), CHUNK)],
                                    feat_v[p], sem_f[p])

    issue_feat(0)
    issue_feat(1)

    pltpu.sync_copy(w_hbm.at[pl.ds(sid * WROWS, WROWS)], wtile_v)

    def sig_body(r, carry):
        for c in range(D // L):
            w = wtile_v[r, pl.ds(c * L, L)]
            wtile_v[r, pl.ds(c * L, L)] = 1.0 / (1.0 + jnp.exp(-w))
        return carry

    lax.fori_loop(0, WROWS, sig_body, 0)
    pltpu.sync_copy(wtile_v, sig_sh.at[pl.ds(sid * WROWS, WROWS)])
    plsc.subcore_barrier()

    gathers = {}

    def issue_gather(j):
        p = j % NBUF
        gathers[j] = pltpu.async_copy(sig_sh.at[idx_v.at[j]], rows_v[p], sem_g[p])

    issue_gather(0)
    issue_gather(1)

    for j in range(NCHUNK):
        p = j % NBUF
        gathers[j].wait()
        feats[j].wait()
        if j >= NBUF:
            outs[j - NBUF].wait()

        def row_body(r, carry):
            for c in range(D // L):
                s = rows_v[p][r, pl.ds(c * L, L)]
                f = feat_v[p][r, pl.ds(c * L, L)]
                out_v[p][r, pl.ds(c * L, L)] = f * s
            return carry

        lax.fori_loop(0, CHUNK, row_body, 0)
        outs[j] = pltpu.async_copy(out_v[p], out_hbm.at[pl.ds(row0_of(j), CHUNK)],
                                   sem_o[p])
        if j + NBUF < NCHUNK:
            issue_feat(j + NBUF)
            issue_gather(j + NBUF)

    for j in range(NCHUNK - NBUF, NCHUNK):
        outs[j].wait()


def kernel(features, labels, W):
    lab = labels.astype(jnp.int32).reshape(NW, NCHUNK, CHUNK)
    w_pad = jnp.pad(W, ((0, CP - C), (0, 0)))
    return _sc_weighter(features, lab, w_pad)
